# baseline (speedup 1.0000x reference)
import functools

import jax
import jax.numpy as jnp
from jax import lax
from jax.experimental import pallas as pl
from jax.experimental.pallas import tpu as pltpu
from jax.experimental.pallas import tpu_sc as plsc

_NUM_BINS = 100000
_ROWS, _COLS = 16384, 100
_TOTAL = _ROWS * _COLS
_NC, _NS, _L = 2, 16, 16
_NW = _NC * _NS
_PER_W = _TOTAL // _NW
_NVEC = _PER_W // _L

def _hash16(x):
    c = jnp.uint32(0x45D9F3B)
    x = (x ^ (x >> 16)) * c
    x = (x ^ (x >> 16)) * c
    h = x ^ (x >> 16)
    xp = h >> 5
    a = xp >> 16
    b = xp & jnp.uint32(0xFFFF)
    mh = jnp.uint32(2684)
    ml = jnp.uint32(23237)
    mid = a * ml + b * mh + ((b * ml) >> 16)
    q = (a * mh + (mid >> 16)) >> 7
    return (h - q * jnp.uint32(_NUM_BINS)).astype(jnp.int32)


@functools.partial(
    pl.kernel,
    out_type=jax.ShapeDtypeStruct((_TOTAL,), jnp.int32),
    mesh=plsc.VectorSubcoreMesh(core_axis_name="c", subcore_axis_name="s"),
    scratch_types=[
        pltpu.VMEM((_PER_W,), jnp.int32),
        pltpu.VMEM((_PER_W,), jnp.int32),
    ],
)
def _hash_sc(x_hbm, out_hbm, in_v, out_v):
    wid = lax.axis_index("s") * _NC + lax.axis_index("c")
    base = wid * _PER_W
    pltpu.sync_copy(x_hbm.at[pl.ds(base, _PER_W)], in_v)

    @pl.loop(0, _NVEC, unroll=8)
    def _(i):
        x = in_v[pl.ds(i * _L, _L)].astype(jnp.uint32)
        out_v[pl.ds(i * _L, _L)] = _hash16(x)

    pltpu.sync_copy(out_v, out_hbm.at[pl.ds(base, _PER_W)])


def kernel(inputs):
    flat = inputs.reshape(_TOTAL)
    return _hash_sc(flat).reshape(_ROWS, _COLS)


# speedup vs baseline: 3.7797x; 3.7797x over previous
import functools

import jax
import jax.numpy as jnp
from jax.experimental import pallas as pl

_NUM_BINS = 100000
_ROWS, _COLS = 16384, 100
_BR = 1024
_GRID = _ROWS // _BR


def _hash_mod(x):
    c = jnp.uint32(0x45D9F3B)
    x = (x ^ (x >> 16)) * c
    x = (x ^ (x >> 16)) * c
    h = x ^ (x >> 16)
    xp = h >> 5
    a = xp >> 16
    b = xp & jnp.uint32(0xFFFF)
    mh = jnp.uint32(2684)
    ml = jnp.uint32(23237)
    mid = a * ml + b * mh + ((b * ml) >> 16)
    q = (a * mh + (mid >> 16)) >> 7
    return (h - q * jnp.uint32(_NUM_BINS)).astype(jnp.int32)


def _body(x_ref, o_ref):
    o_ref[...] = _hash_mod(x_ref[...].astype(jnp.uint32))


@jax.jit
def _tc_hash(x):
    spec = pl.BlockSpec((_BR, _COLS), lambda i: (i, 0))
    return pl.pallas_call(
        _body,
        grid=(_GRID,),
        in_specs=[spec],
        out_specs=spec,
        out_shape=jax.ShapeDtypeStruct((_ROWS, _COLS), jnp.int32),
    )(x)


def kernel(inputs):
    return _tc_hash(inputs)


# speedup vs baseline: 11.8983x; 3.1480x over previous
import jax
import jax.numpy as jnp
from jax.experimental import pallas as pl
from jax.experimental.pallas import tpu as pltpu

_NUM_BINS = 100000
_ROWS, _COLS = 16384, 100
_BC = 2048
_GRID = _ROWS // _BC


def _hash_mod(x):
    c = jnp.uint32(0x45D9F3B)
    x = (x ^ (x >> 16)) * c
    x = (x ^ (x >> 16)) * c
    h = x ^ (x >> 16)
    return (h % jnp.uint32(_NUM_BINS)).astype(jnp.int32)


def _body(x_ref, o_ref):
    o_ref[...] = _hash_mod(x_ref[...].astype(jnp.uint32))


def _tc_hash_t(xt):
    spec = pl.BlockSpec((_COLS, _BC), lambda i: (0, i))
    return pl.pallas_call(
        _body,
        grid=(_GRID,),
        in_specs=[spec],
        out_specs=spec,
        out_shape=jax.ShapeDtypeStruct((_COLS, _ROWS), jnp.int32),
        compiler_params=pltpu.CompilerParams(
            dimension_semantics=("parallel",)),
    )(xt)


def kernel(inputs):
    return _tc_hash_t(inputs.T).T
